# Initial kernel scaffold; baseline (speedup 1.0000x reference)
import functools

import jax
import jax.numpy as jnp
from jax import lax
from jax.experimental import pallas as pl
from jax.experimental.pallas import tpu as pltpu
from jax.experimental.pallas import tpu_sc as plsc

D = 32
LANES = 16
NC = 2
NS = 16
NW = NC * NS
CHUNK = 800


def _rsqrt16(x):
    i = plsc.bitcast(x, jnp.int32)
    i = jnp.int32(0x5F3759DF) - lax.shift_right_arithmetic(i, 1)
    y = plsc.bitcast(i, jnp.float32)
    for _ in range(3):
        y = y * (1.5 - 0.5 * x * y * y)
    return y


@functools.cache
def _gather_norm(n_rows):
    per_w = n_rows // NW
    n_chunks = per_w // CHUNK
    assert per_w % CHUNK == 0 and CHUNK % LANES == 0
    mesh = plsc.VectorSubcoreMesh(core_axis_name="c", subcore_axis_name="s")

    @functools.partial(
        pl.kernel,
        out_type=jax.ShapeDtypeStruct((n_rows, D), jnp.float32),
        mesh=mesh,
        scratch_types=[
            pltpu.VMEM((CHUNK,), jnp.int32),
            pltpu.VMEM((CHUNK, D), jnp.float32),
            pltpu.SemaphoreType.DMA,
        ],
    )
    def k(table_hbm, idx_hbm, out_hbm, idx_v, rows_v, sem):
        wid = lax.axis_index("s") * NC + lax.axis_index("c")
        w_base = wid * per_w

        def chunk_body(ci, carry):
            base = w_base + ci * CHUNK
            pltpu.sync_copy(idx_hbm.at[pl.ds(base, CHUNK)], idx_v)
            pltpu.async_copy(table_hbm.at[idx_v], rows_v, sem).wait()

            def grp_body(g, c2):
                rows16 = lax.iota(jnp.int32, 16) + g * LANES
                acc = jnp.zeros((LANES,), jnp.float32)
                cols = []
                for d in range(D):
                    cd = jnp.full((LANES,), d, jnp.int32)
                    col = plsc.load_gather(rows_v, [rows16, cd])
                    cols.append(col)
                    acc = acc + col * col
                r = _rsqrt16(jnp.maximum(acc, 1e-24))
                for d in range(D):
                    cd = jnp.full((LANES,), d, jnp.int32)
                    plsc.store_scatter(rows_v, [rows16, cd], cols[d] * r)
                return c2

            lax.fori_loop(0, CHUNK // LANES, grp_body, 0)
            pltpu.sync_copy(rows_v, out_hbm.at[pl.ds(base, CHUNK)])
            return carry

        lax.fori_loop(0, n_chunks, chunk_body, 0)

    return k


def _rel_proj(edge_attr, r_weight):
    e, k = edge_attr.shape
    block = 12800

    def mm(x_ref, w_ref, o_ref):
        o_ref[...] = lax.dot_general(
            x_ref[...], w_ref[...],
            (((1,), (1,)), ((), ())),
            preferred_element_type=jnp.float32,
        )

    return pl.pallas_call(
        mm,
        grid=(e // block,),
        in_specs=[
            pl.BlockSpec((block, k), lambda i: (i, 0)),
            pl.BlockSpec((D, k), lambda i: (0, 0)),
        ],
        out_specs=pl.BlockSpec((block, D), lambda i: (i, 0)),
        out_shape=jax.ShapeDtypeStruct((e, D), jnp.float32),
    )(edge_attr, r_weight)


def kernel(edge_index, edge_attr, entity_table, r_weight):
    e = edge_index.shape[1]
    idx_flat = edge_index.reshape(2 * e)
    flat = _gather_norm(2 * e)(entity_table, idx_flat)
    edge_emb = flat.reshape(2, e, D)
    rel_emb = _rel_proj(edge_attr, r_weight)
    return (edge_emb, rel_emb)


# speedup vs baseline: 1.5379x; 1.5379x over previous
import functools

import jax
import jax.numpy as jnp
from jax import lax
from jax.experimental import pallas as pl
from jax.experimental.pallas import tpu as pltpu
from jax.experimental.pallas import tpu_sc as plsc

D = 32
LANES = 16
NC = 2
NS = 16
NW = NC * NS
CHUNK = 800


def _rsqrt16(x):
    i = plsc.bitcast(x, jnp.int32)
    i = jnp.int32(0x5F3759DF) - lax.shift_right_arithmetic(i, 1)
    y = plsc.bitcast(i, jnp.float32)
    for _ in range(3):
        y = y * (1.5 - 0.5 * x * y * y)
    return y


@functools.cache
def _gather_norm(n_rows):
    per_w = n_rows // NW
    n_chunks = per_w // CHUNK
    assert per_w % CHUNK == 0 and CHUNK % LANES == 0
    mesh = plsc.VectorSubcoreMesh(core_axis_name="c", subcore_axis_name="s")

    @functools.partial(
        pl.kernel,
        out_type=jax.ShapeDtypeStruct((n_rows, D), jnp.float32),
        mesh=mesh,
        scratch_types=[
            pltpu.VMEM((CHUNK,), jnp.int32),
            pltpu.VMEM((CHUNK, D), jnp.float32),
            pltpu.SemaphoreType.DMA,
        ],
        compiler_params=pltpu.CompilerParams(
            use_tc_tiling_on_sc=False, needs_layout_passes=False
        ),
    )
    def k(table_hbm, idx_hbm, out_hbm, idx_v, rows_v, sem):
        wid = lax.axis_index("s") * NC + lax.axis_index("c")
        w_base = wid * per_w

        def chunk_body(ci, carry):
            base = w_base + ci * CHUNK
            pltpu.sync_copy(idx_hbm.at[pl.ds(base, CHUNK)], idx_v)
            pltpu.async_copy(table_hbm.at[idx_v], rows_v, sem).wait()

            def grp_body(g, c2):
                rows16 = lax.iota(jnp.int32, 16) + g * LANES
                acc = jnp.zeros((LANES,), jnp.float32)
                cols = []
                for d in range(D):
                    cd = jnp.full((LANES,), d, jnp.int32)
                    col = plsc.load_gather(rows_v, [rows16, cd])
                    cols.append(col)
                    acc = acc + col * col
                r = _rsqrt16(jnp.maximum(acc, 1e-24))
                for d in range(D):
                    cd = jnp.full((LANES,), d, jnp.int32)
                    plsc.store_scatter(rows_v, [rows16, cd], cols[d] * r)
                return c2

            lax.fori_loop(0, CHUNK // LANES, grp_body, 0)
            pltpu.sync_copy(rows_v, out_hbm.at[pl.ds(base, CHUNK)])
            return carry

        lax.fori_loop(0, n_chunks, chunk_body, 0)

    return k


def _rel_proj(edge_attr, r_weight):
    e, k = edge_attr.shape
    block = 12800

    def mm(x_ref, w_ref, o_ref):
        o_ref[...] = lax.dot_general(
            x_ref[...], w_ref[...],
            (((1,), (1,)), ((), ())),
            preferred_element_type=jnp.float32,
        )

    return pl.pallas_call(
        mm,
        grid=(e // block,),
        in_specs=[
            pl.BlockSpec((block, k), lambda i: (i, 0)),
            pl.BlockSpec((D, k), lambda i: (0, 0)),
        ],
        out_specs=pl.BlockSpec((block, D), lambda i: (i, 0)),
        out_shape=jax.ShapeDtypeStruct((e, D), jnp.float32),
    )(edge_attr, r_weight)


def kernel(edge_index, edge_attr, entity_table, r_weight):
    e = edge_index.shape[1]
    idx_flat = edge_index.reshape(2 * e)
    flat = _gather_norm(2 * e)(entity_table, idx_flat)
    edge_emb = flat.reshape(2, e, D)
    rel_emb = _rel_proj(edge_attr, r_weight)
    return (edge_emb, rel_emb)


# speedup vs baseline: 1.9833x; 1.2896x over previous
import functools

import jax
import jax.numpy as jnp
from jax import lax
from jax.experimental import pallas as pl
from jax.experimental.pallas import tpu as pltpu
from jax.experimental.pallas import tpu_sc as plsc

D = 32
LANES = 16
NC = 2
NS = 16
NW = NC * NS
CHUNK = 400


def _rsqrt16(x):
    i = plsc.bitcast(x, jnp.int32)
    i = jnp.int32(0x5F3759DF) - lax.shift_right_arithmetic(i, 1)
    y = plsc.bitcast(i, jnp.float32)
    for _ in range(3):
        y = y * (1.5 - 0.5 * x * y * y)
    return y


@functools.cache
def _gather_norm(e):
    per_w = 2 * e // NW
    n_chunks = per_w // CHUNK
    assert per_w % CHUNK == 0 and CHUNK % LANES == 0 and n_chunks % 2 == 0
    n_groups = CHUNK // LANES
    mesh = plsc.VectorSubcoreMesh(core_axis_name="c", subcore_axis_name="s")

    @functools.partial(
        pl.kernel,
        out_type=jax.ShapeDtypeStruct((2, e, D), jnp.float32),
        mesh=mesh,
        scratch_types=[
            pltpu.VMEM((CHUNK,), jnp.int32),
            pltpu.VMEM((CHUNK,), jnp.int32),
            pltpu.VMEM((CHUNK, D), jnp.float32),
            pltpu.VMEM((CHUNK, D), jnp.float32),
            pltpu.SemaphoreType.DMA,
            pltpu.SemaphoreType.DMA,
            pltpu.SemaphoreType.DMA,
            pltpu.SemaphoreType.DMA,
            pltpu.SemaphoreType.DMA,
            pltpu.SemaphoreType.DMA,
        ],
        compiler_params=pltpu.CompilerParams(
            use_tc_tiling_on_sc=False, needs_layout_passes=False
        ),
    )
    def k(eidx_hbm, table_hbm, out_hbm, i0, i1, r0, r1,
          si0, si1, sg0, sg1, sw0, sw1):
        idx_v = (i0, i1)
        rows_v = (r0, r1)
        sem_i = (si0, si1)
        sem_g = (sg0, sg1)
        sem_w = (sw0, sw1)
        wid = lax.axis_index("c") * NS + lax.axis_index("s")
        half = wid // (NW // 2)
        wrow = (wid % (NW // 2)) * per_w

        def idx_copy(i, b):
            return pltpu.make_async_copy(
                eidx_hbm.at[half, pl.ds(wrow + i * CHUNK, CHUNK)],
                idx_v[b], sem_i[b])

        def gather_copy(b):
            return pltpu.make_async_copy(
                table_hbm.at[idx_v[b]], rows_v[b], sem_g[b])

        def wb_copy(i, b):
            return pltpu.make_async_copy(
                rows_v[b],
                out_hbm.at[half, pl.ds(wrow + i * CHUNK, CHUNK)],
                sem_w[b])

        def normalize(b):
            def grp_body(g, c2):
                rows16 = lax.iota(jnp.int32, LANES) + g * LANES
                acc = jnp.zeros((LANES,), jnp.float32)
                cols = []
                for d in range(D):
                    cd = jnp.full((LANES,), d, jnp.int32)
                    col = plsc.load_gather(rows_v[b], [rows16, cd])
                    cols.append(col)
                    acc = acc + col * col
                r = _rsqrt16(jnp.maximum(acc, 1e-24))
                for d in range(D):
                    cd = jnp.full((LANES,), d, jnp.int32)
                    plsc.store_scatter(rows_v[b], [rows16, cd], cols[d] * r)
                return c2

            lax.fori_loop(0, n_groups, grp_body, 0)

        idx_copy(0, 0).start()
        idx_copy(1, 1).start()
        idx_copy(0, 0).wait()
        gather_copy(0).start()

        def superstep(ss, carry):
            for b in (0, 1):
                i = 2 * ss + b
                gather_copy(b).wait()

                @pl.when(i + 2 < n_chunks)
                def _():
                    idx_copy(i + 2, b).start()

                @pl.when(i >= 1)
                def _():
                    wb_copy(i - 1, 1 - b).wait()

                @pl.when(i + 1 < n_chunks)
                def _():
                    idx_copy(i + 1, 1 - b).wait()
                    gather_copy(1 - b).start()

                normalize(b)
                wb_copy(i, b).start()
            return carry

        lax.fori_loop(0, n_chunks // 2, superstep, 0)
        wb_copy(n_chunks - 1, 1).wait()

    return k


def _rel_proj(edge_attr, r_weight):
    e = edge_attr.shape[0]
    e8 = e // 8
    block = 2000

    wt = jnp.tile(r_weight.T, (8, 8))
    pa = lax.broadcasted_iota(jnp.int32, (128, 256), 0) // 16
    qa = lax.broadcasted_iota(jnp.int32, (128, 256), 1) // D
    bigw = jnp.where(pa == qa, wt, 0.0)

    x2 = edge_attr.reshape(e8, 128)

    def mm(x_ref, w_ref, o_ref):
        o_ref[...] = jnp.dot(
            x_ref[...], w_ref[...], preferred_element_type=jnp.float32
        )

    out2 = pl.pallas_call(
        mm,
        grid=(e8 // block,),
        in_specs=[
            pl.BlockSpec((block, 128), lambda i: (i, 0)),
            pl.BlockSpec((128, 256), lambda i: (0, 0)),
        ],
        out_specs=pl.BlockSpec((block, 8 * D), lambda i: (i, 0)),
        out_shape=jax.ShapeDtypeStruct((e8, 8 * D), jnp.float32),
    )(x2, bigw)
    return out2.reshape(e, D)


def kernel(edge_index, edge_attr, entity_table, r_weight):
    e = edge_index.shape[1]
    edge_emb = _gather_norm(e)(edge_index, entity_table)
    rel_emb = _rel_proj(edge_attr, r_weight)
    return (edge_emb, rel_emb)
